# baseline (speedup 1.0000x reference)
import jax
import jax.numpy as jnp
import numpy as np
from jax.experimental import pallas as pl
from jax.experimental.pallas import tpu as pltpu

_OUT = 32
_PROJ = _OUT * (_OUT + 1) // 2
_FLAT = _OUT * _OUT


def _sym_perm() -> np.ndarray:
    rows, cols = np.triu_indices(_OUT)
    m = np.zeros((_OUT, _OUT), dtype=np.int32)
    m[rows, cols] = np.arange(_PROJ, dtype=np.int32)
    m[cols, rows] = np.arange(_PROJ, dtype=np.int32)
    return m.reshape(-1)


_G = _sym_perm()


def _proj_kernel(w_ref, x_ref, b_ref, o_ref):
    o_ref[...] = (
        jax.lax.dot_general(
            w_ref[...],
            x_ref[...],
            (((1,), (1,)), ((), ())),
            preferred_element_type=jnp.float32,
        )
        + b_ref[...]
    )


def kernel(node_feats, W, b):
    n, d = node_feats.shape
    w2 = W[_G].astype(jnp.float32)
    b2 = b[_G][:, None].astype(jnp.float32)

    bn = 12800
    bk = 128
    grid = ((n + bn - 1) // bn, _FLAT // bk)

    out_t = pl.pallas_call(
        _proj_kernel,
        grid=grid,
        in_specs=[
            pl.BlockSpec((bk, d), lambda i, j: (j, 0)),
            pl.BlockSpec((bn, d), lambda i, j: (i, 0)),
            pl.BlockSpec((bk, 1), lambda i, j: (j, 0)),
        ],
        out_specs=pl.BlockSpec((bk, bn), lambda i, j: (j, i)),
        out_shape=jax.ShapeDtypeStruct((_FLAT, n), jnp.float32),
        compiler_params=pltpu.CompilerParams(
            dimension_semantics=("parallel", "parallel")
        ),
    )(w2, node_feats, b2)
    return out_t.reshape(_OUT, _OUT, n).transpose(2, 0, 1)


# speedup vs baseline: 1.0918x; 1.0918x over previous
import jax
import jax.numpy as jnp
import numpy as np
from jax.experimental import pallas as pl
from jax.experimental.pallas import tpu as pltpu

_OUT = 32
_PROJ = _OUT * (_OUT + 1) // 2
_FLAT = _OUT * _OUT


def _sym_perm() -> np.ndarray:
    rows, cols = np.triu_indices(_OUT)
    m = np.zeros((_OUT, _OUT), dtype=np.int32)
    m[rows, cols] = np.arange(_PROJ, dtype=np.int32)
    m[cols, rows] = np.arange(_PROJ, dtype=np.int32)
    return m.reshape(-1)


_G = _sym_perm()


def _proj_kernel(w_ref, x_ref, b_ref, o_ref):
    o_ref[...] = (
        jax.lax.dot_general(
            w_ref[...],
            x_ref[...],
            (((1,), (1,)), ((), ())),
            preferred_element_type=jnp.float32,
        )
        + b_ref[...]
    )


def kernel(node_feats, W, b):
    n, d = node_feats.shape
    w2 = W[_G].astype(jnp.float32)
    b2 = b[_G][:, None].astype(jnp.float32)

    bn = 5120
    grid = (n + bn - 1) // bn

    out_t = pl.pallas_call(
        _proj_kernel,
        grid=(grid,),
        in_specs=[
            pl.BlockSpec((_FLAT, d), lambda i: (0, 0)),
            pl.BlockSpec((bn, d), lambda i: (i, 0)),
            pl.BlockSpec((_FLAT, 1), lambda i: (0, 0)),
        ],
        out_specs=pl.BlockSpec((_FLAT, bn), lambda i: (0, i)),
        out_shape=jax.ShapeDtypeStruct((_FLAT, n), jnp.float32),
        compiler_params=pltpu.CompilerParams(
            dimension_semantics=("parallel",)
        ),
    )(w2, node_feats, b2)
    return out_t.reshape(_OUT, _OUT, n).transpose(2, 0, 1)


# speedup vs baseline: 1.0927x; 1.0008x over previous
import jax
import jax.numpy as jnp
import numpy as np
from jax.experimental import pallas as pl
from jax.experimental.pallas import tpu as pltpu

_OUT = 32
_PROJ = _OUT * (_OUT + 1) // 2
_FLAT = _OUT * _OUT


def _sym_perm() -> np.ndarray:
    rows, cols = np.triu_indices(_OUT)
    m = np.zeros((_OUT, _OUT), dtype=np.int32)
    m[rows, cols] = np.arange(_PROJ, dtype=np.int32)
    m[cols, rows] = np.arange(_PROJ, dtype=np.int32)
    return m.reshape(-1)


_G = _sym_perm()


def _proj_kernel(w_ref, x_ref, b_ref, o_ref):
    o_ref[...] = (
        jax.lax.dot_general(
            w_ref[...],
            x_ref[...],
            (((1,), (1,)), ((), ())),
            preferred_element_type=jnp.float32,
        )
        + b_ref[...]
    )


def kernel(node_feats, W, b):
    n, d = node_feats.shape
    w2 = W[_G].astype(jnp.float32)
    b2 = b[_G][:, None].astype(jnp.float32)

    bn = 4096
    grid = (n + bn - 1) // bn

    out_t = pl.pallas_call(
        _proj_kernel,
        grid=(grid,),
        in_specs=[
            pl.BlockSpec((_FLAT, d), lambda i: (0, 0)),
            pl.BlockSpec((bn, d), lambda i: (i, 0)),
            pl.BlockSpec((_FLAT, 1), lambda i: (0, 0)),
        ],
        out_specs=pl.BlockSpec((_FLAT, bn), lambda i: (0, i)),
        out_shape=jax.ShapeDtypeStruct((_FLAT, n), jnp.float32),
        compiler_params=pltpu.CompilerParams(
            dimension_semantics=("parallel",)
        ),
    )(w2, node_feats, b2)
    return out_t.reshape(_OUT, _OUT, n).transpose(2, 0, 1)
